# Initial kernel scaffold; baseline (speedup 1.0000x reference)
import jax
import jax.numpy as jnp
from jax.experimental import pallas as pl


def kernel(x, edge_index, edge_weight):
    raise NotImplementedError("write your pallas kernel here")


# speedup vs baseline: 2.1455x; 2.1455x over previous
import functools

import jax
import jax.numpy as jnp
from jax import lax
from jax.experimental import pallas as pl
from jax.experimental.pallas import tpu as pltpu
from jax.experimental.pallas import tpu_sc as plsc

K = 10
ALPHA = 0.1
NW = 32
NSUB = 16
LANES = 16
CH = 80
EB = 128
NP_ = 10240
D = 128

_mesh = plsc.VectorSubcoreMesh(core_axis_name="c", subcore_axis_name="s")
f32 = jnp.float32
i32 = jnp.int32


def _wid():
    return lax.axis_index("c") * NSUB + lax.axis_index("s")


@functools.partial(
    pl.kernel,
    out_type=jax.ShapeDtypeStruct((2 * NP_,), f32),
    mesh=_mesh,
    scratch_types=[
        pltpu.VMEM_SHARED((NP_,), f32),
        pltpu.VMEM((CH, EB), i32),
        pltpu.VMEM((CH, EB), f32),
        pltpu.VMEM((640,), f32),
    ],
)
def _deg_kernel(rowr, wgtr, degp_out, deg_sh, row_v, w_v, zbuf):
    cid = lax.axis_index("c")
    sid = lax.axis_index("s")
    eb = cid * NSUB + sid

    @pl.loop(0, 40)
    def _z(i):
        zbuf[pl.ds(i * LANES, LANES)] = jnp.zeros((LANES,), f32)

    pltpu.sync_copy(zbuf, deg_sh.at[pl.ds(sid * 640, 640)])
    plsc.subcore_barrier()

    pltpu.sync_copy(rowr.at[eb], row_v)
    pltpu.sync_copy(wgtr.at[eb], w_v)

    @pl.loop(0, CH)
    def _scatter(j):
        pltpu.sync_copy(w_v.at[j], deg_sh.at[row_v.at[j]], add=True)

    plsc.subcore_barrier()
    pltpu.sync_copy(
        deg_sh.at[pl.ds(sid * 640, 640)],
        degp_out.at[pl.ds(cid * NP_ + sid * 640, 640)],
    )


@functools.partial(
    pl.kernel,
    out_type=(
        jax.ShapeDtypeStruct((NP_,), f32),
        jax.ShapeDtypeStruct((NP_, D), f32),
    ),
    mesh=_mesh,
    scratch_types=[
        pltpu.VMEM((320,), f32),
        pltpu.VMEM((320,), f32),
        pltpu.VMEM((64, D), f32),
    ],
)
def _prep_kernel(degp, x_in, deg_out, xph_out, da, db, xbuf):
    w = _wid()
    base = w * 320
    pltpu.sync_copy(degp.at[pl.ds(base, 320)], da)
    pltpu.sync_copy(degp.at[pl.ds(NP_ + base, 320)], db)

    @pl.loop(0, 20)
    def _add(i):
        sl = pl.ds(i * LANES, LANES)
        da[sl] = da[sl] + db[sl]

    pltpu.sync_copy(da, deg_out.at[pl.ds(base, 320)])

    @pl.loop(0, 5)
    def _xc(c):
        rb = base + c * 64
        pltpu.sync_copy(x_in.at[pl.ds(rb, 64)], xbuf)

        @pl.loop(0, 64)
        def _row(r):
            for f in range(D // LANES):
                sl = pl.ds(f * LANES, LANES)
                xbuf[r, sl] = xbuf[r, sl] * (ALPHA * 0.5)

        pltpu.sync_copy(xbuf, xph_out.at[pl.ds(rb, 64)])


@functools.partial(
    pl.kernel,
    out_type=jax.ShapeDtypeStruct((NW, CH, EB), f32),
    mesh=_mesh,
    scratch_types=[
        pltpu.VMEM((CH, EB), i32),
        pltpu.VMEM((CH, EB), f32),
        pltpu.VMEM((EB,), f32),
    ],
)
def _norm_kernel(rowr, wgtr, deg_in, norm_out, row_v, w_v, dbuf):
    eb = _wid()
    pltpu.sync_copy(rowr.at[eb], row_v)
    pltpu.sync_copy(wgtr.at[eb], w_v)

    @pl.loop(0, CH)
    def _chunk(j):
        pltpu.sync_copy(deg_in.at[row_v.at[j]], dbuf)
        for g in range(EB // LANES):
            sl = pl.ds(g * LANES, LANES)
            w_v[j, sl] = ((1.0 - ALPHA) * w_v[j, sl]) / (dbuf[sl] + 1e-10)

    pltpu.sync_copy(w_v, norm_out.at[eb])


@functools.partial(
    pl.kernel,
    out_type=(
        jax.ShapeDtypeStruct((2 * NP_, D), f32),
        jax.ShapeDtypeStruct((NP_, D), f32),
    ),
    mesh=_mesh,
    scratch_types=[
        pltpu.VMEM_SHARED((NP_, D), f32),
        pltpu.VMEM((EB,), i32),
        pltpu.VMEM((EB,), i32),
        pltpu.VMEM((EB,), f32),
        pltpu.VMEM((EB, D), f32),
        pltpu.VMEM((EB, D), f32),
        pltpu.SemaphoreType.DMA,
        pltpu.SemaphoreType.DMA,
    ],
)
def _step_kernel(po, xph, colr, rowr, normr, po_out, h_out,
                 msg_sh, col_c, row_c, norm_c, rows0, rows1, sem0, sem1):
    cid = lax.axis_index("c")
    sid = lax.axis_index("s")
    eb = cid * NSUB + sid
    nbase = sid * 640

    @pl.loop(0, 5)
    def _comb(c):
        rb = nbase + c * EB
        pltpu.sync_copy(po.at[pl.ds(rb, EB)], rows0)
        pltpu.sync_copy(po.at[pl.ds(NP_ + rb, EB)], rows1)

        @pl.loop(0, EB)
        def _row(r):
            for f in range(D // LANES):
                sl = pl.ds(f * LANES, LANES)
                rows0[r, sl] = rows0[r, sl] + rows1[r, sl]

        pltpu.sync_copy(rows0, h_out.at[pl.ds(rb, EB)])

    pltpu.sync_copy(xph.at[pl.ds(nbase, 640)], msg_sh.at[pl.ds(nbase, 640)])

    plsc.subcore_barrier()

    @pl.loop(0, CH)
    def _edge_chunk(j):
        pltpu.sync_copy(colr.at[eb, j], col_c)
        pltpu.sync_copy(rowr.at[eb, j], row_c)
        pltpu.sync_copy(normr.at[eb, j], norm_c)
        pltpu.async_copy(h_out.at[col_c], rows0, sem0).wait()

        @pl.loop(0, EB // LANES)
        def _scale(g):
            nv = norm_c[pl.ds(g * LANES, LANES)]
            for i in range(LANES):
                e = g * LANES + i
                s = nv[i]
                for f in range(D // LANES):
                    sl = pl.ds(f * LANES, LANES)
                    rows0[e, sl] = rows0[e, sl] * s

        pltpu.sync_copy(rows0, msg_sh.at[row_c], add=True)

    plsc.subcore_barrier()

    pltpu.sync_copy(
        msg_sh.at[pl.ds(nbase, 640)],
        po_out.at[pl.ds(cid * NP_ + nbase, 640)],
    )


@functools.partial(
    pl.kernel,
    out_type=jax.ShapeDtypeStruct((NP_, D), f32),
    mesh=_mesh,
    scratch_types=[
        pltpu.VMEM((64, D), f32),
        pltpu.VMEM((64, D), f32),
    ],
)
def _final_kernel(po, h_out, ba, bb):
    w = _wid()
    base = w * 320

    @pl.loop(0, 5)
    def _comb(c):
        rb = base + c * 64
        pltpu.sync_copy(po.at[pl.ds(rb, 64)], ba)
        pltpu.sync_copy(po.at[pl.ds(NP_ + rb, 64)], bb)

        @pl.loop(0, 64)
        def _row(r):
            for f in range(D // LANES):
                sl = pl.ds(f * LANES, LANES)
                ba[r, sl] = ba[r, sl] + bb[r, sl]

        pltpu.sync_copy(ba, h_out.at[pl.ds(rb, 64)])


def kernel(x, edge_index, edge_weight):
    n = x.shape[0]
    e = edge_weight.shape[0]
    ep = NW * CH * EB

    row = edge_index[0].astype(i32)
    col = edge_index[1].astype(i32)
    w = edge_weight.astype(f32)

    rowr = jnp.pad(row, (0, ep - e)).reshape(NW, CH, EB)
    colr = jnp.pad(col, (0, ep - e)).reshape(NW, CH, EB)
    wgtr = jnp.pad(w, (0, ep - e)).reshape(NW, CH, EB)
    x_p = jnp.pad(x.astype(f32), ((0, NP_ - n), (0, 0)))

    degp = _deg_kernel(rowr, wgtr)
    deg, xph = _prep_kernel(degp, x_p)
    normr = _norm_kernel(rowr, wgtr, deg)

    po = jnp.concatenate([x_p, jnp.zeros_like(x_p)], axis=0)
    for _ in range(K):
        po, _h = _step_kernel(po, xph, colr, rowr, normr)
    h = _final_kernel(po)
    return h[:n]
